# baseline (speedup 1.0000x reference)
import functools

import jax
import jax.numpy as jnp
from jax import lax
from jax.experimental import pallas as pl
from jax.experimental.pallas import tpu as pltpu
from jax.experimental.pallas import tpu_sc as plsc

_NPOINT = 1024
_RADIUS = 0.2
_NSAMPLE = 32
_SUB = 8
_PQ = 64
_PM = 128
_DP = 128
_CHUNK = 128


def _fps_body(coor_ref, out_ref, *, n, npoint, lanes, nb):
    xs = [coor_ref[b, 0] for b in range(nb)]
    ys = [coor_ref[b, 1] for b in range(nb)]
    zs = [coor_ref[b, 2] for b in range(nb)]
    rows = lax.broadcasted_iota(jnp.int32, (_SUB, lanes), 0)
    cols = lax.broadcasted_iota(jnp.int32, (_SUB, lanes), 1)
    fiota = rows * lanes + cols

    def step(k, carry):
        dists, fars = carry
        ndists, nfars = [], []
        for b in range(nb):
            sel = fiota == fars[b]
            cx = jnp.max(jnp.where(sel, xs[b], -jnp.inf))
            cy = jnp.max(jnp.where(sel, ys[b], -jnp.inf))
            cz = jnp.max(jnp.where(sel, zs[b], -jnp.inf))
            out_ref[b, pl.ds(k, 1), :] = jnp.concatenate(
                [cx.reshape(1, 1), cy.reshape(1, 1), cz.reshape(1, 1)], axis=1)
            dx = xs[b] - cx
            dy = ys[b] - cy
            dz = zs[b] - cz
            d = (dx * dx + dy * dy) + dz * dz
            dist = jnp.minimum(dists[b], d)
            m = jnp.max(dist)
            nf = jnp.min(jnp.where(dist == m, fiota, n))
            ndists.append(dist)
            nfars.append(nf)
        return tuple(ndists), tuple(nfars)

    dist0 = jnp.full((_SUB, lanes), jnp.inf, dtype=jnp.float32)
    lax.fori_loop(0, npoint, step,
                  (tuple(dist0 for _ in range(nb)),
                   tuple(jnp.int32(0) for _ in range(nb))))


def _fps_centers(points_coor):
    b, _, n = points_coor.shape
    lanes = n // _SUB
    coor4 = points_coor.reshape(b, 3, _SUB, lanes)
    nbp = 2
    body = functools.partial(_fps_body, n=n, npoint=_NPOINT, lanes=lanes,
                             nb=nbp)
    return pl.pallas_call(
        body,
        grid=(b // nbp,),
        in_specs=[pl.BlockSpec((nbp, 3, _SUB, lanes), lambda i: (i, 0, 0, 0))],
        out_specs=pl.BlockSpec((nbp, _NPOINT, 3), lambda i: (i, 0, 0)),
        out_shape=jax.ShapeDtypeStruct((b, _NPOINT, 3), jnp.float32),
    )(coor4)


_CAP = 768
_COFF = 736
_CPP = 8


def _scq_body(coor_hbm, cen_hbm, cd_hbm, ci_hbm, xr, yr, zr, cxr, cyr, czr,
              cdb, cib, sem, *, n, r2, nw, cpt, nb, tot):
    wid = lax.axis_index("s") * 2 + lax.axis_index("c")
    bi = wid // (nw // nb)
    pltpu.sync_copy(coor_hbm.at[pl.ds((bi * 3 + 0) * n, n)], xr)
    pltpu.sync_copy(coor_hbm.at[pl.ds((bi * 3 + 1) * n, n)], yr)
    pltpu.sync_copy(coor_hbm.at[pl.ds((bi * 3 + 2) * n, n)], zr)
    pltpu.sync_copy(cen_hbm.at[pl.ds((0 * tot + wid * cpt) * 16, cpt * 16)], cxr)
    pltpu.sync_copy(cen_hbm.at[pl.ds((1 * tot + wid * cpt) * 16, cpt * 16)], cyr)
    pltpu.sync_copy(cen_hbm.at[pl.ds((2 * tot + wid * cpt) * 16, cpt * 16)], czr)

    lane = lax.broadcasted_iota(jnp.int32, (16,), 0)

    def b16(v):
        bits = plsc.bitcast(v, jnp.int32)
        r = (bits + (jnp.int32(32767) + ((bits >> 16) & 1))) & jnp.int32(-65536)
        return plsc.bitcast(r, jnp.float32)

    pend = []
    for p in range(cpt // _CPP):
        sl = p % 2
        if len(pend) >= 2:
            for h in pend[-2]:
                h.wait()
        def initb(i, carry):
            cdb[pl.ds(sl * _CPP * _CAP + i * 16, 16)] = jnp.full(
                (16,), jnp.inf, jnp.float32)
            cib[pl.ds(sl * _CPP * _CAP + i * 16, 16)] = jnp.zeros(
                (16,), jnp.int32)
            return carry
        lax.fori_loop(0, _CPP * _CAP // 16, initb, 0)

        qxs, qys, qzs, qns = [], [], [], []
        for c in range(_CPP):
            g16 = (p * _CPP + c) * 16
            qx = cxr[pl.ds(g16, 16)]
            qy = cyr[pl.ds(g16, 16)]
            qz = czr[pl.ds(g16, 16)]
            qxs.append(b16(qx))
            qys.append(b16(qy))
            qzs.append(b16(qz))
            qns.append((qx * qx + qy * qy) + qz * qz)

        def scan_pts(i, offs):
            xv = xr[pl.ds(i * 16, 16)]
            yv = yr[pl.ds(i * 16, 16)]
            zv = zr[pl.ds(i * 16, 16)]
            pnv = (xv * xv + yv * yv) + zv * zv
            xvb = b16(xv)
            yvb = b16(yv)
            zvb = b16(zv)
            idxv = lane + i * 16
            new_offs = []
            for c in range(_CPP):
                dot = (qxs[c] * xvb + qys[c] * yvb) + qzs[c] * zvb
                d = (qns[c] + pnv) - 2.0 * dot
                msk = d <= r2
                plsc.store_compressed(
                    cdb.at[pl.ds((sl * _CPP + c) * _CAP + offs[c], 16)],
                    d, mask=msk)
                plsc.store_compressed(
                    cib.at[pl.ds((sl * _CPP + c) * _CAP + offs[c], 16)],
                    idxv, mask=msk)
                pcv = plsc.all_reduce_population_count(msk)
                new_offs.append(jnp.minimum(offs[c] + pcv[0], _COFF))
            return tuple(new_offs)

        lax.fori_loop(0, n // 16, scan_pts,
                      tuple(jnp.int32(0) for _ in range(_CPP)))

        row0 = wid * cpt + p * _CPP
        h1 = pltpu.async_copy(
            cdb.at[pl.ds(sl * _CPP * _CAP, _CPP * _CAP)],
            cd_hbm.at[pl.ds(row0 * _CAP, _CPP * _CAP)], sem)
        h2 = pltpu.async_copy(
            cib.at[pl.ds(sl * _CPP * _CAP, _CPP * _CAP)],
            ci_hbm.at[pl.ds(row0 * _CAP, _CPP * _CAP)], sem)
        pend.append((h1, h2))

    for hs in pend[-2:]:
        for h in hs:
            h.wait()


def _sc_query(coor_flat, cen_splat_flat, n, tot, nb):
    b = nb
    nw = 32
    cpt = tot // nw
    mesh = plsc.VectorSubcoreMesh(core_axis_name="c", subcore_axis_name="s")
    body = functools.partial(_scq_body, n=n, r2=_RADIUS * _RADIUS, nw=nw,
                             cpt=cpt, nb=b, tot=tot)
    f = pl.kernel(
        body, mesh=mesh,
        compiler_params=pltpu.CompilerParams(needs_layout_passes=False),
        out_type=(jax.ShapeDtypeStruct((tot * _CAP,), jnp.float32),
                  jax.ShapeDtypeStruct((tot * _CAP,), jnp.int32)),
        scratch_types=[
            pltpu.VMEM((n,), jnp.float32),
            pltpu.VMEM((n,), jnp.float32),
            pltpu.VMEM((n,), jnp.float32),
            pltpu.VMEM((cpt * 16,), jnp.float32),
            pltpu.VMEM((cpt * 16,), jnp.float32),
            pltpu.VMEM((cpt * 16,), jnp.float32),
            pltpu.VMEM((2 * _CPP * _CAP,), jnp.float32),
            pltpu.VMEM((2 * _CPP * _CAP,), jnp.int32),
            pltpu.SemaphoreType.DMA,
        ],
    )
    return f(coor_flat, cen_splat_flat)


def _select_body(cd_ref, ci_ref, gidx_ref, sc_ref, *, boff, ns, r2, rows):
    sc_ref[...] = cd_ref[...]
    pos = lax.broadcasted_iota(jnp.int32, (rows, _CAP), 1)
    lanes32 = lax.broadcasted_iota(jnp.int32, (rows, ns), 1)
    civ = ci_ref[...]

    def step(k, carry):
        idx_acc, isel0 = carry
        s = sc_ref[...]
        m = jnp.min(s, axis=1, keepdims=True)
        selpos = jnp.min(jnp.where(s == m, pos, _CAP), axis=1, keepdims=True)
        eq2 = pos == selpos
        isel = jnp.max(jnp.where(eq2, civ, 0), axis=1, keepdims=True)
        sc_ref[...] = jnp.where(eq2, jnp.inf, s)
        isel0 = jnp.where(k == 0, isel, isel0)
        chosen = jnp.where(m <= r2, isel, isel0)
        idx_acc = jnp.where(lanes32 == k, chosen, idx_acc)
        return idx_acc, isel0

    idx0 = jnp.zeros((rows, ns), jnp.int32)
    isel0 = jnp.zeros((rows, 1), jnp.int32)
    idx_acc, _ = lax.fori_loop(0, ns, step, (idx0, isel0))
    gidx_ref[...] = idx_acc + boff


def _select_topk(cd, ci, boff):
    tot = cd.shape[0]
    rows = 128
    body = functools.partial(_select_body, boff=boff, ns=_NSAMPLE,
                             r2=_RADIUS * _RADIUS, rows=rows)
    return pl.pallas_call(
        body,
        grid=(tot // rows,),
        in_specs=[
            pl.BlockSpec((rows, _CAP), lambda i: (i, 0)),
            pl.BlockSpec((rows, _CAP), lambda i: (i, 0)),
        ],
        out_specs=pl.BlockSpec((rows, _NSAMPLE), lambda i: (i, 0)),
        out_shape=jax.ShapeDtypeStruct((tot, _NSAMPLE), jnp.int32),
        scratch_shapes=[pltpu.VMEM((rows, _CAP), jnp.float32)],
    )(cd, ci)


def _query_topk(points_coor, centers_k3):
    b, _, n = points_coor.shape
    npoint = centers_k3.shape[1]
    cen_splat = jnp.repeat(
        centers_k3.transpose(0, 2, 1).reshape(b, 3, npoint, 1), 16, axis=3)
    coor_flat = points_coor.reshape(b, 3 * n)
    outs = []
    for bi in range(b):
        cd, ci = _sc_query(coor_flat[bi], cen_splat[bi].reshape(-1),
                           n, npoint, 1)
        gidx = _select_topk(cd.reshape(npoint, _CAP),
                            ci.reshape(npoint, _CAP), bi * n)
        outs.append(gidx)
    return jnp.stack(outs).reshape(b, npoint, _NSAMPLE)


def _gather_rows(table, flat_idx):
    tot = flat_idx.shape[0]
    nw = 32
    per_w = tot // nw
    nchunk = per_w // _CHUNK
    idx3 = flat_idx.reshape(nw, nchunk, _CHUNK)
    mesh = plsc.VectorSubcoreMesh(core_axis_name="c", subcore_axis_name="s")

    @functools.partial(
        pl.kernel, mesh=mesh,
        out_type=jax.ShapeDtypeStruct((tot, _DP), jnp.float32),
        scratch_types=[
            pltpu.VMEM((nchunk, _CHUNK), jnp.int32),
            pltpu.VMEM((_CHUNK, _DP), jnp.float32),
            pltpu.SemaphoreType.DMA,
        ],
    )
    def gather_k(table_hbm, idx_hbm, out_hbm, idx_v, rows_v, sem):
        wid = lax.axis_index("s") * 2 + lax.axis_index("c")
        base = wid * per_w
        pltpu.sync_copy(idx_hbm.at[wid], idx_v)

        def chunk(j, carry):
            pltpu.async_copy(table_hbm.at[idx_v.at[j]], rows_v, sem).wait()
            pltpu.sync_copy(rows_v, out_hbm.at[pl.ds(base + j * _CHUNK, _CHUNK)])
            return carry

        lax.fori_loop(0, nchunk, chunk, 0)

    return gather_k(table, idx3)


def _mlp_body(g_ref, q_ref, w_ref, b_ref, ga_ref, be_ref, out_ref, *, c_out):
    pm = q_ref.shape[1]
    g = g_ref[0]
    q = q_ref[0]
    lane = lax.broadcasted_iota(jnp.int32, (1, _DP), 1)
    scale = jnp.where(lane < 64, 1.0, jnp.where(lane < 67, 1.0 / _RADIUS, 0.0))
    cen = jnp.concatenate(
        [jnp.zeros((pm, 64), jnp.float32), q * (1.0 / _RADIUS),
         jnp.zeros((pm, _DP - 67), jnp.float32)], axis=1)
    gs = g * scale[None] - cen[:, None, :]
    g2 = gs.reshape(pm * _NSAMPLE, _DP)
    h = jnp.dot(g2, w_ref[...], preferred_element_type=jnp.float32) + b_ref[...]
    mu = jnp.mean(h, axis=-1, keepdims=True)
    var = jnp.mean((h - mu) ** 2, axis=-1, keepdims=True)
    h = (h - mu) / jnp.sqrt(var + 1e-5) * ga_ref[...] + be_ref[...]
    h = jnp.maximum(h, 0.0)
    out_ref[0] = jnp.max(h.reshape(pm, _NSAMPLE, c_out), axis=1)


def _mlp_pool(gathered, centers_k3, W, b, gamma, beta):
    bsz, npoint = gathered.shape[:2]
    c_out, c_in = W.shape
    wp = jnp.zeros((_DP, c_out), W.dtype).at[:c_in, :].set(W.T)
    body = functools.partial(_mlp_body, c_out=c_out)
    return pl.pallas_call(
        body,
        grid=(bsz, npoint // _PM),
        in_specs=[
            pl.BlockSpec((1, _PM, _NSAMPLE, _DP), lambda i, j: (i, j, 0, 0)),
            pl.BlockSpec((1, _PM, 3), lambda i, j: (i, j, 0)),
            pl.BlockSpec((_DP, c_out), lambda i, j: (0, 0)),
            pl.BlockSpec((1, c_out), lambda i, j: (0, 0)),
            pl.BlockSpec((1, c_out), lambda i, j: (0, 0)),
            pl.BlockSpec((1, c_out), lambda i, j: (0, 0)),
        ],
        out_specs=pl.BlockSpec((1, _PM, c_out), lambda i, j: (i, j, 0)),
        out_shape=jax.ShapeDtypeStruct((bsz, npoint, c_out), jnp.float32),
    )(gathered, centers_k3, wp, b.reshape(1, -1), gamma.reshape(1, -1),
      beta.reshape(1, -1))


def kernel(points_coor, points_fea, points_padding, W, b, gamma, beta):
    bsz, _, n = points_coor.shape
    c_in = points_fea.shape[1]

    centers_k3 = _fps_centers(points_coor)
    gidx = _query_topk(points_coor, centers_k3)

    fea_t = jnp.transpose(points_fea, (0, 2, 1))
    coor_t = jnp.transpose(points_coor, (0, 2, 1))
    table = jnp.concatenate(
        [fea_t, coor_t,
         jnp.zeros((bsz, n, _DP - c_in - 3), jnp.float32)], axis=-1
    ).reshape(bsz * n, _DP)

    gathered = _gather_rows(table, gidx.reshape(-1))
    gathered = gathered.reshape(bsz, _NPOINT, _NSAMPLE, _DP)

    new_fea = _mlp_pool(gathered, centers_k3, W, b, gamma, beta)

    new_mask = jnp.zeros((bsz, _NPOINT), dtype=bool)
    return (jnp.transpose(centers_k3, (0, 2, 1)),
            jnp.transpose(new_fea, (0, 2, 1)),
            new_mask)


# speedup vs baseline: 1.1598x; 1.1598x over previous
import functools

import jax
import jax.numpy as jnp
from jax import lax
from jax.experimental import pallas as pl
from jax.experimental.pallas import tpu as pltpu
from jax.experimental.pallas import tpu_sc as plsc

_NPOINT = 1024
_RADIUS = 0.2
_NSAMPLE = 32
_SUB = 8
_PQ = 64
_PM = 128
_DP = 128
_CHUNK = 128


def _fps_body(coor_ref, csm_ref, out_ref, *, n, npoint, lanes, nb):
    xs = [coor_ref[b, 0] for b in range(nb)]
    ys = [coor_ref[b, 1] for b in range(nb)]
    zs = [coor_ref[b, 2] for b in range(nb)]
    rows = lax.broadcasted_iota(jnp.int32, (_SUB, lanes), 0)
    cols = lax.broadcasted_iota(jnp.int32, (_SUB, lanes), 1)
    fiota = rows * lanes + cols

    def step(k, carry):
        dists, fars = carry
        ndists, nfars = [], []
        for b in range(nb):
            cx = csm_ref[b, 0, fars[b]]
            cy = csm_ref[b, 1, fars[b]]
            cz = csm_ref[b, 2, fars[b]]
            out_ref[b, pl.ds(k, 1), :] = jnp.concatenate(
                [cx.reshape(1, 1), cy.reshape(1, 1), cz.reshape(1, 1)], axis=1)
            dx = xs[b] - cx
            dy = ys[b] - cy
            dz = zs[b] - cz
            d = (dx * dx + dy * dy) + dz * dz
            dist = jnp.minimum(dists[b], d)
            m = jnp.max(dist)
            nf = jnp.min(jnp.where(dist == m, fiota, n))
            ndists.append(dist)
            nfars.append(nf)
        return tuple(ndists), tuple(nfars)

    dist0 = jnp.full((_SUB, lanes), jnp.inf, dtype=jnp.float32)
    lax.fori_loop(0, npoint, step,
                  (tuple(dist0 for _ in range(nb)),
                   tuple(jnp.int32(0) for _ in range(nb))))


def _fps_centers(points_coor):
    b, _, n = points_coor.shape
    lanes = n // _SUB
    coor4 = points_coor.reshape(b, 3, _SUB, lanes)
    nbp = 1
    body = functools.partial(_fps_body, n=n, npoint=_NPOINT, lanes=lanes,
                             nb=nbp)
    return pl.pallas_call(
        body,
        grid=(b // nbp,),
        in_specs=[
            pl.BlockSpec((nbp, 3, _SUB, lanes), lambda i: (i, 0, 0, 0)),
            pl.BlockSpec((nbp, 3, n), lambda i: (i, 0, 0),
                         memory_space=pltpu.SMEM),
        ],
        out_specs=pl.BlockSpec((nbp, _NPOINT, 3), lambda i: (i, 0, 0)),
        out_shape=jax.ShapeDtypeStruct((b, _NPOINT, 3), jnp.float32),
    )(coor4, points_coor)


_CAP = 768
_COFF = 736
_CPP = 16


def _scq_body(coor_hbm, cen_hbm, cd_hbm, ci_hbm, xr, yr, zr, cxr, cyr, czr,
              cdb, cib, sem, *, n, r2, nw, cpt, nb, tot):
    wid = lax.axis_index("s") * 2 + lax.axis_index("c")
    bi = wid // (nw // nb)
    pltpu.sync_copy(coor_hbm.at[pl.ds((bi * 3 + 0) * n, n)], xr)
    pltpu.sync_copy(coor_hbm.at[pl.ds((bi * 3 + 1) * n, n)], yr)
    pltpu.sync_copy(coor_hbm.at[pl.ds((bi * 3 + 2) * n, n)], zr)
    pltpu.sync_copy(cen_hbm.at[pl.ds((0 * tot + wid * cpt) * 16, cpt * 16)], cxr)
    pltpu.sync_copy(cen_hbm.at[pl.ds((1 * tot + wid * cpt) * 16, cpt * 16)], cyr)
    pltpu.sync_copy(cen_hbm.at[pl.ds((2 * tot + wid * cpt) * 16, cpt * 16)], czr)

    lane = lax.broadcasted_iota(jnp.int32, (16,), 0)

    def b16(v):
        bits = plsc.bitcast(v, jnp.int32)
        r = (bits + (jnp.int32(32767) + ((bits >> 16) & 1))) & jnp.int32(-65536)
        return plsc.bitcast(r, jnp.float32)

    pend = []
    for p in range(cpt // _CPP):
        sl = p % 2
        if len(pend) >= 2:
            for h in pend[-2]:
                h.wait()
        def initb(i, carry):
            cdb[pl.ds(sl * _CPP * _CAP + i * 16, 16)] = jnp.full(
                (16,), jnp.inf, jnp.float32)
            cib[pl.ds(sl * _CPP * _CAP + i * 16, 16)] = jnp.zeros(
                (16,), jnp.int32)
            return carry
        lax.fori_loop(0, _CPP * _CAP // 16, initb, 0)

        qxs, qys, qzs, qns = [], [], [], []
        for c in range(_CPP):
            g16 = (p * _CPP + c) * 16
            qx = cxr[pl.ds(g16, 16)]
            qy = cyr[pl.ds(g16, 16)]
            qz = czr[pl.ds(g16, 16)]
            qxs.append(b16(qx))
            qys.append(b16(qy))
            qzs.append(b16(qz))
            qns.append((qx * qx + qy * qy) + qz * qz)

        def scan_pts(i, offs):
            xv = xr[pl.ds(i * 16, 16)]
            yv = yr[pl.ds(i * 16, 16)]
            zv = zr[pl.ds(i * 16, 16)]
            pnv = (xv * xv + yv * yv) + zv * zv
            xvb = b16(xv)
            yvb = b16(yv)
            zvb = b16(zv)
            idxv = lane + i * 16
            new_offs = []
            for c in range(_CPP):
                dot = (qxs[c] * xvb + qys[c] * yvb) + qzs[c] * zvb
                d = (qns[c] + pnv) - 2.0 * dot
                msk = d <= r2
                plsc.store_compressed(
                    cdb.at[pl.ds((sl * _CPP + c) * _CAP + offs[c], 16)],
                    d, mask=msk)
                plsc.store_compressed(
                    cib.at[pl.ds((sl * _CPP + c) * _CAP + offs[c], 16)],
                    idxv, mask=msk)
                pcv = plsc.all_reduce_population_count(msk)
                new_offs.append(jnp.minimum(offs[c] + pcv[0], _COFF))
            return tuple(new_offs)

        lax.fori_loop(0, n // 16, scan_pts,
                      tuple(jnp.int32(0) for _ in range(_CPP)))

        row0 = wid * cpt + p * _CPP
        h1 = pltpu.async_copy(
            cdb.at[pl.ds(sl * _CPP * _CAP, _CPP * _CAP)],
            cd_hbm.at[pl.ds(row0 * _CAP, _CPP * _CAP)], sem)
        h2 = pltpu.async_copy(
            cib.at[pl.ds(sl * _CPP * _CAP, _CPP * _CAP)],
            ci_hbm.at[pl.ds(row0 * _CAP, _CPP * _CAP)], sem)
        pend.append((h1, h2))

    for hs in pend[-2:]:
        for h in hs:
            h.wait()


def _sc_query(coor_flat, cen_splat_flat, n, tot, nb):
    b = nb
    nw = 32
    cpt = tot // nw
    mesh = plsc.VectorSubcoreMesh(core_axis_name="c", subcore_axis_name="s")
    body = functools.partial(_scq_body, n=n, r2=_RADIUS * _RADIUS, nw=nw,
                             cpt=cpt, nb=b, tot=tot)
    f = pl.kernel(
        body, mesh=mesh,
        compiler_params=pltpu.CompilerParams(needs_layout_passes=False),
        out_type=(jax.ShapeDtypeStruct((tot * _CAP,), jnp.float32),
                  jax.ShapeDtypeStruct((tot * _CAP,), jnp.int32)),
        scratch_types=[
            pltpu.VMEM((n,), jnp.float32),
            pltpu.VMEM((n,), jnp.float32),
            pltpu.VMEM((n,), jnp.float32),
            pltpu.VMEM((cpt * 16,), jnp.float32),
            pltpu.VMEM((cpt * 16,), jnp.float32),
            pltpu.VMEM((cpt * 16,), jnp.float32),
            pltpu.VMEM((2 * _CPP * _CAP,), jnp.float32),
            pltpu.VMEM((2 * _CPP * _CAP,), jnp.int32),
            pltpu.SemaphoreType.DMA,
        ],
    )
    return f(coor_flat, cen_splat_flat)


def _select_body(cd_ref, ci_ref, gidx_ref, sc_ref, *, boff, ns, r2, rows):
    sc_ref[...] = cd_ref[...]
    pos = lax.broadcasted_iota(jnp.int32, (rows, _CAP), 1)
    lanes32 = lax.broadcasted_iota(jnp.int32, (rows, ns), 1)
    civ = ci_ref[...]

    def step(k, carry):
        idx_acc, isel0 = carry
        s = sc_ref[...]
        m = jnp.min(s, axis=1, keepdims=True)
        selpos = jnp.min(jnp.where(s == m, pos, _CAP), axis=1, keepdims=True)
        eq2 = pos == selpos
        isel = jnp.max(jnp.where(eq2, civ, 0), axis=1, keepdims=True)
        sc_ref[...] = jnp.where(eq2, jnp.inf, s)
        isel0 = jnp.where(k == 0, isel, isel0)
        chosen = jnp.where(m <= r2, isel, isel0)
        idx_acc = jnp.where(lanes32 == k, chosen, idx_acc)
        return idx_acc, isel0

    idx0 = jnp.zeros((rows, ns), jnp.int32)
    isel0 = jnp.zeros((rows, 1), jnp.int32)
    idx_acc, _ = lax.fori_loop(0, ns, step, (idx0, isel0))
    gidx_ref[...] = idx_acc + boff


def _select_topk(cd, ci, boff):
    tot = cd.shape[0]
    rows = 128
    body = functools.partial(_select_body, boff=boff, ns=_NSAMPLE,
                             r2=_RADIUS * _RADIUS, rows=rows)
    return pl.pallas_call(
        body,
        grid=(tot // rows,),
        in_specs=[
            pl.BlockSpec((rows, _CAP), lambda i: (i, 0)),
            pl.BlockSpec((rows, _CAP), lambda i: (i, 0)),
        ],
        out_specs=pl.BlockSpec((rows, _NSAMPLE), lambda i: (i, 0)),
        out_shape=jax.ShapeDtypeStruct((tot, _NSAMPLE), jnp.int32),
        scratch_shapes=[pltpu.VMEM((rows, _CAP), jnp.float32)],
    )(cd, ci)


def _query_topk(points_coor, centers_k3):
    b, _, n = points_coor.shape
    npoint = centers_k3.shape[1]
    cen_splat = jnp.repeat(
        centers_k3.transpose(0, 2, 1).reshape(b, 3, npoint, 1), 16, axis=3)
    coor_flat = points_coor.reshape(b, 3 * n)
    outs = []
    for bi in range(b):
        cd, ci = _sc_query(coor_flat[bi], cen_splat[bi].reshape(-1),
                           n, npoint, 1)
        gidx = _select_topk(cd.reshape(npoint, _CAP),
                            ci.reshape(npoint, _CAP), bi * n)
        outs.append(gidx)
    return jnp.stack(outs).reshape(b, npoint, _NSAMPLE)


def _gather_rows(table, flat_idx):
    tot = flat_idx.shape[0]
    nw = 32
    per_w = tot // nw
    nchunk = per_w // _CHUNK
    idx3 = flat_idx.reshape(nw, nchunk, _CHUNK)
    mesh = plsc.VectorSubcoreMesh(core_axis_name="c", subcore_axis_name="s")

    @functools.partial(
        pl.kernel, mesh=mesh,
        out_type=jax.ShapeDtypeStruct((tot, _DP), jnp.float32),
        scratch_types=[
            pltpu.VMEM((nchunk, _CHUNK), jnp.int32),
            pltpu.VMEM((_CHUNK, _DP), jnp.float32),
            pltpu.SemaphoreType.DMA,
        ],
    )
    def gather_k(table_hbm, idx_hbm, out_hbm, idx_v, rows_v, sem):
        wid = lax.axis_index("s") * 2 + lax.axis_index("c")
        base = wid * per_w
        pltpu.sync_copy(idx_hbm.at[wid], idx_v)

        def chunk(j, carry):
            pltpu.async_copy(table_hbm.at[idx_v.at[j]], rows_v, sem).wait()
            pltpu.sync_copy(rows_v, out_hbm.at[pl.ds(base + j * _CHUNK, _CHUNK)])
            return carry

        lax.fori_loop(0, nchunk, chunk, 0)

    return gather_k(table, idx3)


def _mlp_body(g_ref, q_ref, w_ref, b_ref, ga_ref, be_ref, out_ref, *, c_out):
    pm = q_ref.shape[1]
    g = g_ref[0]
    q = q_ref[0]
    lane = lax.broadcasted_iota(jnp.int32, (1, _DP), 1)
    scale = jnp.where(lane < 64, 1.0, jnp.where(lane < 67, 1.0 / _RADIUS, 0.0))
    cen = jnp.concatenate(
        [jnp.zeros((pm, 64), jnp.float32), q * (1.0 / _RADIUS),
         jnp.zeros((pm, _DP - 67), jnp.float32)], axis=1)
    gs = g * scale[None] - cen[:, None, :]
    g2 = gs.reshape(pm * _NSAMPLE, _DP)
    h = jnp.dot(g2, w_ref[...], preferred_element_type=jnp.float32) + b_ref[...]
    mu = jnp.mean(h, axis=-1, keepdims=True)
    var = jnp.mean((h - mu) ** 2, axis=-1, keepdims=True)
    h = (h - mu) / jnp.sqrt(var + 1e-5) * ga_ref[...] + be_ref[...]
    h = jnp.maximum(h, 0.0)
    out_ref[0] = jnp.max(h.reshape(pm, _NSAMPLE, c_out), axis=1)


def _mlp_pool(gathered, centers_k3, W, b, gamma, beta):
    bsz, npoint = gathered.shape[:2]
    c_out, c_in = W.shape
    wp = jnp.zeros((_DP, c_out), W.dtype).at[:c_in, :].set(W.T)
    body = functools.partial(_mlp_body, c_out=c_out)
    return pl.pallas_call(
        body,
        grid=(bsz, npoint // _PM),
        in_specs=[
            pl.BlockSpec((1, _PM, _NSAMPLE, _DP), lambda i, j: (i, j, 0, 0)),
            pl.BlockSpec((1, _PM, 3), lambda i, j: (i, j, 0)),
            pl.BlockSpec((_DP, c_out), lambda i, j: (0, 0)),
            pl.BlockSpec((1, c_out), lambda i, j: (0, 0)),
            pl.BlockSpec((1, c_out), lambda i, j: (0, 0)),
            pl.BlockSpec((1, c_out), lambda i, j: (0, 0)),
        ],
        out_specs=pl.BlockSpec((1, _PM, c_out), lambda i, j: (i, j, 0)),
        out_shape=jax.ShapeDtypeStruct((bsz, npoint, c_out), jnp.float32),
    )(gathered, centers_k3, wp, b.reshape(1, -1), gamma.reshape(1, -1),
      beta.reshape(1, -1))


def kernel(points_coor, points_fea, points_padding, W, b, gamma, beta):
    bsz, _, n = points_coor.shape
    c_in = points_fea.shape[1]

    centers_k3 = _fps_centers(points_coor)
    gidx = _query_topk(points_coor, centers_k3)

    fea_t = jnp.transpose(points_fea, (0, 2, 1))
    coor_t = jnp.transpose(points_coor, (0, 2, 1))
    table = jnp.concatenate(
        [fea_t, coor_t,
         jnp.zeros((bsz, n, _DP - c_in - 3), jnp.float32)], axis=-1
    ).reshape(bsz * n, _DP)

    gathered = _gather_rows(table, gidx.reshape(-1))
    gathered = gathered.reshape(bsz, _NPOINT, _NSAMPLE, _DP)

    new_fea = _mlp_pool(gathered, centers_k3, W, b, gamma, beta)

    new_mask = jnp.zeros((bsz, _NPOINT), dtype=bool)
    return (jnp.transpose(centers_k3, (0, 2, 1)),
            jnp.transpose(new_fea, (0, 2, 1)),
            new_mask)
